# baseline (speedup 1.0000x reference)
import jax
import jax.numpy as jnp
from jax import lax
from jax.experimental import pallas as pl
from jax.experimental.pallas import tpu as pltpu

B, T, DS = 4, 8192, 1024
N, DR = 256, 1024
HID = 128
K = 32
ROWS = 1024
NSTEPS = (B * T) // ROWS
SPB = T // ROWS
RCHUNK = (B * N) // NSTEPS
NTILES = ROWS // 8


def _body(s_ref, r_ref, wst_ref, wct_ref, w1t_ref, b1_ref, w2p_ref, b2_ref,
          out_ref, acc_ref, rw_ref):
    t = pl.program_id(0)
    b = t // SPB

    @pl.when(t % SPB == 0)
    def _init():
        acc_ref[pl.ds(8 * b, 8), :] = jnp.zeros((8, DS), jnp.float32)

    a = acc_ref[pl.ds(8 * b, 8), :]
    for k in range(NTILES):
        a = a + s_ref[8 * k:8 * k + 8, :]
    acc_ref[pl.ds(8 * b, 8), :] = a

    rw_ref[pl.ds(t * RCHUNK, RCHUNK), :] = lax.dot_general(
        r_ref[...], wst_ref[...], (((1,), (0,)), ((), ())),
        preferred_element_type=jnp.float32)

    @pl.when(t == NSTEPS - 1)
    def _tail():
        ctx_rows = []
        for bb in range(B):
            a8 = acc_ref[pl.ds(8 * bb, 8), :]
            r4 = a8[0:4, :] + a8[4:8, :]
            r2 = r4[0:2, :] + r4[2:4, :]
            ctx_rows.append(r2[0:1, :] + r2[1:2, :])
        context = jnp.concatenate(ctx_rows, axis=0) * (1.0 / T)
        ctx = lax.dot_general(context, wct_ref[...],
                              (((1,), (0,)), ((), ())),
                              preferred_element_type=jnp.float32)
        b2 = b2_ref[0, 0]
        for bb in range(B):
            rwb = rw_ref[pl.ds(bb * N, N), :]
            h = jnp.tanh(rwb + ctx[bb:bb + 1, :])
            hid = jnp.maximum(
                lax.dot_general(h, w1t_ref[...], (((1,), (0,)), ((), ())),
                                preferred_element_type=jnp.float32)
                + b1_ref[...], 0.0)
            scp = lax.dot_general(hid, w2p_ref[...], (((1,), (0,)), ((), ())),
                                  preferred_element_type=jnp.float32)
            sc_col = scp[:, 0:1] + b2
            col = lax.broadcast_in_dim(sc_col, (N, N), (0, 1))
            row = lax.transpose(col, (1, 0))
            ii = lax.broadcasted_iota(jnp.int32, (N, N), 0)
            jj = lax.broadcasted_iota(jnp.int32, (N, N), 1)
            beats = (col > row) | ((col == row) & (ii < jj))
            rank = jnp.sum(beats.astype(jnp.int32), axis=0, keepdims=True)
            out_ref[pl.ds(bb, 1), :] = (rank < K).astype(jnp.float32)


def _allocate(srows, r2, wst, wct, w1t, b1r, w2p, b2r, interpret=False):
    return pl.pallas_call(
        _body,
        grid=(NSTEPS,),
        in_specs=[
            pl.BlockSpec((ROWS, DS), lambda t: (t, 0)),
            pl.BlockSpec((RCHUNK, DR), lambda t: (t, 0)),
            pl.BlockSpec((DR, DR), lambda t: (0, 0)),
            pl.BlockSpec((DS, DR), lambda t: (0, 0)),
            pl.BlockSpec((DR, HID), lambda t: (0, 0)),
            pl.BlockSpec((1, HID), lambda t: (0, 0)),
            pl.BlockSpec((HID, HID), lambda t: (0, 0)),
            pl.BlockSpec((1, 1), lambda t: (0, 0)),
        ],
        out_specs=pl.BlockSpec((B, N), lambda t: (0, 0)),
        out_shape=jax.ShapeDtypeStruct((B, N), jnp.float32),
        scratch_shapes=[
            pltpu.VMEM((8 * B, DS), jnp.float32),
            pltpu.VMEM((B * N, DR), jnp.float32),
        ],
        interpret=interpret,
    )(srows, r2, wst, wct, w1t, b1r, w2p, b2r)


@jax.jit
def kernel(s, r, Wc, Ws, W1, b1, W2, b2):
    w2p = jnp.zeros((HID, HID), jnp.float32).at[:, 0].set(W2[0])
    mask = _allocate(s.reshape(B * T, DS), r.reshape(B * N, DR), Ws.T, Wc.T,
                     W1.T, b1.reshape(1, HID), w2p, b2.reshape(1, 1))
    return mask[..., None]


# speedup vs baseline: 1.1474x; 1.1474x over previous
import jax
import jax.numpy as jnp
from jax import lax
from jax.experimental import pallas as pl
from jax.experimental.pallas import tpu as pltpu

B, T, DS = 4, 8192, 1024
N, DR = 256, 1024
HID = 128
K = 32
ROWS = 2048
NSTEPS = (B * T) // ROWS
SPB = T // ROWS
RCHUNK = (B * N) // NSTEPS
NTILES = ROWS // 8


def _body(s_ref, r_ref, wst_ref, wct_ref, w1t_ref, b1_ref, w2p_ref, b2_ref,
          out_ref, acc_ref, rw_ref):
    t = pl.program_id(0)
    b = t // SPB

    @pl.when(t % SPB == 0)
    def _init():
        acc_ref[pl.ds(8 * b, 8), :] = jnp.zeros((8, DS), jnp.float32)

    a = acc_ref[pl.ds(8 * b, 8), :]
    for k in range(NTILES):
        a = a + s_ref[8 * k:8 * k + 8, :]
    acc_ref[pl.ds(8 * b, 8), :] = a

    rw_ref[pl.ds(t * RCHUNK, RCHUNK), :] = lax.dot_general(
        r_ref[...], wst_ref[...], (((1,), (0,)), ((), ())),
        preferred_element_type=jnp.float32)

    @pl.when(t == NSTEPS - 1)
    def _tail():
        ctx_rows = []
        for bb in range(B):
            a8 = acc_ref[pl.ds(8 * bb, 8), :]
            r4 = a8[0:4, :] + a8[4:8, :]
            r2 = r4[0:2, :] + r4[2:4, :]
            ctx_rows.append(r2[0:1, :] + r2[1:2, :])
        context = jnp.concatenate(ctx_rows, axis=0) * (1.0 / T)
        ctx = lax.dot_general(context, wct_ref[...],
                              (((1,), (0,)), ((), ())),
                              preferred_element_type=jnp.float32)
        b2 = b2_ref[0, 0]
        for bb in range(B):
            rwb = rw_ref[pl.ds(bb * N, N), :]
            h = jnp.tanh(rwb + ctx[bb:bb + 1, :])
            hid = jnp.maximum(
                lax.dot_general(h, w1t_ref[...], (((1,), (0,)), ((), ())),
                                preferred_element_type=jnp.float32)
                + b1_ref[...], 0.0)
            scp = lax.dot_general(hid, w2p_ref[...], (((1,), (0,)), ((), ())),
                                  preferred_element_type=jnp.float32)
            sc_col = scp[:, 0:1] + b2
            col = lax.broadcast_in_dim(sc_col, (N, N), (0, 1))
            row = lax.transpose(col, (1, 0))
            ii = lax.broadcasted_iota(jnp.int32, (N, N), 0)
            jj = lax.broadcasted_iota(jnp.int32, (N, N), 1)
            beats = (col > row) | ((col == row) & (ii < jj))
            rank = jnp.sum(beats.astype(jnp.int32), axis=0, keepdims=True)
            out_ref[pl.ds(bb, 1), :] = (rank < K).astype(jnp.float32)


def _allocate(srows, r2, wst, wct, w1t, b1r, w2p, b2r, interpret=False):
    return pl.pallas_call(
        _body,
        grid=(NSTEPS,),
        in_specs=[
            pl.BlockSpec((ROWS, DS), lambda t: (t, 0)),
            pl.BlockSpec((RCHUNK, DR), lambda t: (t, 0)),
            pl.BlockSpec((DR, DR), lambda t: (0, 0)),
            pl.BlockSpec((DS, DR), lambda t: (0, 0)),
            pl.BlockSpec((DR, HID), lambda t: (0, 0)),
            pl.BlockSpec((1, HID), lambda t: (0, 0)),
            pl.BlockSpec((HID, HID), lambda t: (0, 0)),
            pl.BlockSpec((1, 1), lambda t: (0, 0)),
        ],
        out_specs=pl.BlockSpec((B, N), lambda t: (0, 0)),
        out_shape=jax.ShapeDtypeStruct((B, N), jnp.float32),
        scratch_shapes=[
            pltpu.VMEM((8 * B, DS), jnp.float32),
            pltpu.VMEM((B * N, DR), jnp.float32),
        ],
        interpret=interpret,
    )(srows, r2, wst, wct, w1t, b1r, w2p, b2r)


@jax.jit
def kernel(s, r, Wc, Ws, W1, b1, W2, b2):
    w2p = jnp.zeros((HID, HID), jnp.float32).at[:, 0].set(W2[0])
    mask = _allocate(s.reshape(B * T, DS), r.reshape(B * N, DR), Ws.T, Wc.T,
                     W1.T, b1.reshape(1, HID), w2p, b2.reshape(1, 1))
    return mask[..., None]
